# baseline (speedup 1.0000x reference)
import functools

import jax
import jax.numpy as jnp
from jax import lax
from jax.experimental import pallas as pl
from jax.experimental.pallas import tpu as pltpu
from jax.experimental.pallas import tpu_sc as plsc

N = 10000
E = 160000
EMB = 64
H = 128
VOCAB = 257

NC = 2
NS = 16
NW = NC * NS
CH = 128
E_PAD = -(-E // (NW * CH)) * (NW * CH)
EPW = E_PAD // NW
NPAD = 10240
TPB = NPAD // NW
ACC_R = 384
CAPW = 5248
NSEG = NW * NW
CAPS = 1360
SCN2 = 1024
LC = 80
NLC = 4
LPW = NLC * LC
N_PADL = LPW * NW
BLK = 1000
MASK18 = (1 << 18) - 1


def _mesh():
  return plsc.VectorSubcoreMesh(core_axis_name="c", subcore_axis_name="s")


_SC_PARAMS = pltpu.CompilerParams(use_tc_tiling_on_sc=False)


def _emb_lookup(e0, e1, e2, f_r):
  def body(e0r, e1r, e2r, fr, out, idxb, rowsb, sem):
    c = lax.axis_index("c")
    s = lax.axis_index("s")
    wid = s * NC + c
    base = wid * LPW
    for t, et in enumerate((e0r, e1r, e2r)):
      pltpu.sync_copy(fr.at[t, wid], idxb)
      for j in range(NLC):
        pltpu.async_copy(et.at[idxb.at[j]], rowsb, sem).wait()
        pltpu.sync_copy(rowsb, out.at[t, pl.ds(base + j * LC, LC)])

  kern = pl.kernel(
      body,
      out_type=jax.ShapeDtypeStruct((3, N_PADL, EMB), jnp.float32),
      mesh=_mesh(),
      compiler_params=_SC_PARAMS,
      scratch_types=[
          pltpu.VMEM((NLC, LC), jnp.int32),
          pltpu.VMEM((LC, EMB), jnp.float32),
          pltpu.SemaphoreType.DMA,
      ],
  )
  return kern(e0, e1, e2, f_r)


def _prep_lists(src_flat, dst_flat):
  def body(src_h, dst_h, out_l, out_c, sbuf, dbuf, ring, cbuf, ptrs, offs):
    c = lax.axis_index("c")
    s = lax.axis_index("s")
    w = s * NC + c
    base = w * EPW

    def iinit(b, carry):
      ptrs[b] = 0
      offs[b] = 0
      return carry

    lax.fori_loop(0, NW, iinit, 0)

    def blk(i, carry):
      pltpu.sync_copy(src_h.at[pl.ds(base + i * SCN2, SCN2)], sbuf)
      pltpu.sync_copy(dst_h.at[pl.ds(base + i * SCN2, SCN2)], dbuf)

      def vec(k, carry2):
        sv = sbuf[pl.ds(k * 16, 16)]
        dv = dbuf[pl.ds(k * 16, 16)]
        q6 = lax.shift_right_logical(dv, 6)
        bv = lax.shift_right_logical(q6 * 13108, 16)
        ldv = dv - bv * TPB
        pkv = sv | (ldv << 18)
        for e2 in range(16):
          b = bv[e2]
          pk = pkv[e2]
          p = ptrs[b]
          ring[b, pl.ds(p, 16)] = jnp.broadcast_to(pk, (16,))
          pn = p + 1
          fl = (pn >= CH).astype(jnp.int32)

          @pl.when(pn >= CH)
          def _(b=b):
            o = offs[b]
            pltpu.sync_copy(ring.at[b, pl.ds(0, CH)],
                            out_l.at[b * NW + w, pl.ds(o * CH, CH)])
            ring[b, pl.ds(0, 16)] = ring[b, pl.ds(CH, 16)]
            offs[b] = o + 1

          ptrs[b] = pn - CH * fl
        return carry2

      return lax.fori_loop(0, SCN2 // 16, vec, carry)

    lax.fori_loop(0, EPW // SCN2, blk, 0)

    sink = jnp.full((16,), TPB << 18, jnp.int32)

    def drain(b, carry):
      p = ptrs[b]

      @pl.when(p > 0)
      def _():
        def padv(i, pp):
          ring[b, pl.ds(pp, 16)] = sink
          return pp + 16

        lax.fori_loop(0, (CH + 15 - p) // 16, padv, p)
        o = offs[b]
        pltpu.sync_copy(ring.at[b, pl.ds(0, CH)],
                        out_l.at[b * NW + w, pl.ds(o * CH, CH)])
        offs[b] = o + 1

      return carry

    lax.fori_loop(0, NW, drain, 0)

    def cw(b, carry):
      cbuf[pl.ds(b, 16)] = jnp.broadcast_to(offs[b], (16,))
      return carry

    lax.fori_loop(0, NW, cw, 0)
    pltpu.sync_copy(cbuf.at[pl.ds(0, 48)], out_c.at[w])

  kern = pl.kernel(
      body,
      out_type=(jax.ShapeDtypeStruct((NSEG + 1, CAPW), jnp.int32),
                jax.ShapeDtypeStruct((NW, 48), jnp.int32)),
      mesh=_mesh(),
      compiler_params=_SC_PARAMS,
      scratch_types=[
          pltpu.VMEM((SCN2,), jnp.int32),
          pltpu.VMEM((SCN2,), jnp.int32),
          pltpu.VMEM((NW, CH + 16), jnp.int32),
          pltpu.VMEM((48,), jnp.int32),
          pltpu.SMEM((NW,), jnp.int32),
          pltpu.SMEM((NW,), jnp.int32),
      ],
  )
  return kern(src_flat, dst_flat)


def _seg_sum(D, do_cnt, x, lst, cnts):
  XROWS = x.shape[0]

  def body(*refs):
    if do_cnt:
      (x_r, lst_h, cnt_h, out_s, out_c,
       acc, pbuf, srcb, rows, cntb, schb, sem_g, sem_l, cacc) = refs
    else:
      (x_r, lst_h, cnt_h, out_s,
       acc, pbuf, srcb, rows, cntb, schb, sem_g, sem_l) = refs
    c = lax.axis_index("c")
    s = lax.axis_index("s")
    v = s * NC + c

    zero16 = jnp.zeros((16,), jnp.float32)

    def zacc(i, carry):
      for k in range(D // 16):
        acc[i, pl.ds(k * 16, 16)] = zero16
      return carry

    lax.fori_loop(0, ACC_R, zacc, 0)
    if do_cnt:
      def zacc2(i, carry):
        cacc[i, pl.ds(0, 16)] = zero16
        return carry

      lax.fori_loop(0, ACC_R, zacc2, 0)

    ones16 = jnp.full((16,), 1.0, jnp.float32)

    pltpu.sync_copy(cnt_h, cntb)

    def bw(w2, tot):
      nv = cntb[w2, pl.ds(v, 16)][0]
      row = v * NW + w2

      def app(i, t2):
        schb[pl.ds(t2, 16)] = jnp.broadcast_to(row * 64 + i, (16,))
        return t2 + 1

      return lax.fori_loop(0, nv, app, tot)

    ntot = lax.fori_loop(0, NW, bw, 0)
    dummyv = jnp.full((16,), NSEG * 64, jnp.int32)
    schb[pl.ds(ntot, 16)] = dummyv
    schb[pl.ds(ntot + 16, 16)] = dummyv

    def sched(i):
      return schb[pl.ds(i, 16)][0]

    def issue_list(i, s4):
      enc = sched(i)
      row = lax.shift_right_logical(enc, 6)
      jj = enc & 63
      pltpu.async_copy(lst_h.at[row, pl.ds(jj * CH, CH)], pbuf.at[s4], sem_l)

    def wait_list(s4):
      pltpu.make_async_copy(lst_h.at[0, pl.ds(0, CH)], pbuf.at[s4],
                            sem_l).wait()

    def unpack(s4, s2):
      for k in range(CH // 16):
        srcb[s2, pl.ds(k * 16, 16)] = jnp.minimum(
            pbuf[s4, pl.ds(k * 16, 16)] & MASK18, XROWS - 1)

    def issue_gather(s2):
      pltpu.async_copy(x_r.at[srcb.at[s2]], rows.at[s2], sem_g)

    def wait_gather(s2):
      pltpu.make_async_copy(x_r.at[srcb.at[s2]], rows.at[s2], sem_g).wait()

    def accumulate(s4, s2):
      def group(g, cc):
        pkg = pbuf[s4, pl.ds(g * 16, 16)]
        ldv = jnp.minimum(lax.shift_right_logical(pkg, 18), TPB)
        for e2 in range(16):
          ld = ldv[e2]
          e = g * 16 + e2
          for k in range(D // 16):
            plsc.addupdate(acc.at[ld, pl.ds(k * 16, 16)],
                           rows[s2, e, pl.ds(k * 16, 16)])
          if do_cnt:
            plsc.addupdate(cacc.at[ld, pl.ds(0, 16)], ones16)
        return cc

      lax.fori_loop(0, CH // 16, group, 0)

    issue_list(0, 0)
    issue_list(1, 1)
    wait_list(0)
    unpack(0, 0)
    issue_gather(0)
    issue_list(2, 2)

    def step(idx, carry):
      i1 = idx + 1
      s4n = lax.rem(i1, 4)
      s2n = lax.rem(i1, 2)
      wait_list(s4n)
      unpack(s4n, s2n)
      issue_gather(s2n)
      issue_list(idx + 3, lax.rem(idx + 3, 4))
      wait_gather(lax.rem(idx, 2))
      accumulate(lax.rem(idx, 4), lax.rem(idx, 2))
      return carry

    lax.fori_loop(0, ntot, step, 0)
    wait_gather(lax.rem(ntot, 2))
    wait_list(lax.rem(ntot + 1, 4))
    wait_list(lax.rem(ntot + 2, 4))

    pltpu.sync_copy(acc.at[pl.ds(0, TPB)], out_s.at[pl.ds(v * TPB, TPB)])
    if do_cnt:
      pltpu.sync_copy(cacc.at[pl.ds(0, TPB)], out_c.at[pl.ds(v * TPB, TPB)])

  out_type = [jax.ShapeDtypeStruct((NPAD, D), jnp.float32)]
  scratch = [
      pltpu.VMEM((ACC_R, D), jnp.float32),
      pltpu.VMEM((4, CH), jnp.int32),
      pltpu.VMEM((2, CH), jnp.int32),
      pltpu.VMEM((2, CH, D), jnp.float32),
      pltpu.VMEM((NW, 48), jnp.int32),
      pltpu.VMEM((CAPS,), jnp.int32),
      pltpu.SemaphoreType.DMA,
      pltpu.SemaphoreType.DMA,
  ]
  if do_cnt:
    out_type.append(jax.ShapeDtypeStruct((NPAD, 16), jnp.float32))
    scratch += [pltpu.VMEM((ACC_R, 16), jnp.float32)]

  kern = pl.kernel(body, out_type=tuple(out_type), mesh=_mesh(),
                   compiler_params=_SC_PARAMS, scratch_types=scratch)
  return kern(x, lst, cnts)


def _tc_layer(x, s1, c1, ws, wn, pp):
  Din = x.shape[1]

  def body(xr, sr, cr, wsr, wnr, ppr, hr, gr):
    cnt = cr[...][:, 0:1]
    hn = sr[...] / jnp.maximum(cnt, 1.0)
    r = jnp.dot(xr[...], wsr[...], preferred_element_type=jnp.float32)
    r = r + jnp.dot(hn, wnr[...], preferred_element_type=jnp.float32)
    r = r + ppr[0:1, :]
    r = jnp.where(r > 0, r, ppr[1:2, :] * r)
    r = ppr[2:3, :] * r + ppr[3:4, :]
    hr[...] = r

    @pl.when(pl.program_id(0) == 0)
    def _():
      gr[...] = jnp.zeros_like(gr)

    gr[...] += jnp.sum(r, axis=0, keepdims=True) * (1.0 / N)

  return pl.pallas_call(
      body,
      grid=(N // BLK,),
      in_specs=[
          pl.BlockSpec((BLK, Din), lambda i: (i, 0)),
          pl.BlockSpec((BLK, Din), lambda i: (i, 0)),
          pl.BlockSpec((BLK, 16), lambda i: (i, 0)),
          pl.BlockSpec((Din, H), lambda i: (0, 0)),
          pl.BlockSpec((Din, H), lambda i: (0, 0)),
          pl.BlockSpec((8, H), lambda i: (0, 0)),
      ],
      out_specs=[
          pl.BlockSpec((BLK, H), lambda i: (i, 0)),
          pl.BlockSpec((1, H), lambda i: (0, 0)),
      ],
      out_shape=[
          jax.ShapeDtypeStruct((N, H), jnp.float32),
          jax.ShapeDtypeStruct((1, H), jnp.float32),
      ],
  )(x, s1, c1, ws, wn, pp)


def kernel(feat_h, feat_p, feat_hp, eidx_h, eidx_p, eidx_hp,
           emb_h, emb_p, emb_hp, Ws1, Wn1, b1, a1, g1, be1,
           Ws, Wn, b, a, g, be):
  f = jnp.stack([feat_h, feat_p, feat_hp]).astype(jnp.int32)
  f = jnp.pad(f, ((0, 0), (0, N_PADL - N)))
  f_r = f.reshape(3, NW, NLC, LC)

  srcs, dsts = [], []
  for eidx in (eidx_h, eidx_p, eidx_hp):
    ei = eidx.astype(jnp.int32)
    srcs.append(jnp.pad(ei[0], (0, E_PAD - E)))
    dsts.append(jnp.pad(ei[1], (0, E_PAD - E), constant_values=N))

  x0 = _emb_lookup(emb_h, emb_p, emb_hp, f_r)
  lists = [_prep_lists(srcs[t], dsts[t]) for t in range(3)]

  def pack_params(bb, aa, gg, bee):
    return jnp.concatenate(
        [jnp.stack([bb, aa, gg, bee]), jnp.zeros((4, H), jnp.float32)], axis=0)

  gsums = []
  for t in range(3):
    xt = x0[t]
    lst, cnts = lists[t]
    s1, c1 = _seg_sum(EMB, True, xt, lst, cnts)
    h, gs = _tc_layer(xt, s1, c1, Ws1[t], Wn1[t],
                      pack_params(b1[t], a1[t], g1[t], be1[t]))
    t_gs = [gs]
    for l in range(3):
      (s1,) = _seg_sum(H, False, h, lst, cnts)
      h, gs = _tc_layer(h, s1, c1, Ws[l, t], Wn[l, t],
                        pack_params(b[l, t], a[l, t], g[l, t], be[l, t]))
      t_gs.append(gs)
    gsums.append(t_gs)

  g_vec = jnp.concatenate(
      [gsums[t][l].reshape(H) for t in range(3) for l in range(4)])
  return g_vec.reshape(1, 12 * H)


# speedup vs baseline: 13.5023x; 13.5023x over previous
import functools

import jax
import jax.numpy as jnp
from jax import lax
from jax.experimental import pallas as pl
from jax.experimental.pallas import tpu as pltpu
from jax.experimental.pallas import tpu_sc as plsc

N = 10000
E = 160000
EMB = 64
H = 128
VOCAB = 257

NC = 2
NS = 16
NW = NC * NS
CH = 128
NCHUNK = -(-E // (NW * CH))
EPW = NCHUNK * CH
E_PAD = EPW * NW
NPAD = 10240
RPT = NPAD // NS
LC = 80
NLC = 4
LPW = NLC * LC
N_PADL = LPW * NW
CW = 8
BLK = 1000


def _mesh():
  return plsc.VectorSubcoreMesh(core_axis_name="c", subcore_axis_name="s")


_SC_PARAMS = pltpu.CompilerParams(use_tc_tiling_on_sc=False)


def _emb_lookup(e0, e1, e2, f_r):
  def body(e0r, e1r, e2r, fr, out, idxb, rowsb, sem):
    c = lax.axis_index("c")
    s = lax.axis_index("s")
    wid = s * NC + c
    base = wid * LPW
    for t, et in enumerate((e0r, e1r, e2r)):
      pltpu.sync_copy(fr.at[t, wid], idxb)
      for j in range(NLC):
        pltpu.async_copy(et.at[idxb.at[j]], rowsb, sem).wait()
        pltpu.sync_copy(rowsb, out.at[t, pl.ds(base + j * LC, LC)])

  kern = pl.kernel(
      body,
      out_type=jax.ShapeDtypeStruct((3, N_PADL, EMB), jnp.float32),
      mesh=_mesh(),
      compiler_params=_SC_PARAMS,
      scratch_types=[
          pltpu.VMEM((NLC, LC), jnp.int32),
          pltpu.VMEM((LC, EMB), jnp.float32),
          pltpu.SemaphoreType.DMA,
      ],
  )
  return kern(e0, e1, e2, f_r)


def _seg_sum(D, do_cnt, x, src_r, dst_r, zcnt=None, ones=None):
  def body(*refs):
    if do_cnt:
      (x_r, src_h, dst_h, zc, on, out_s, out_c,
       acc, srcb, dstb, rows, sem, sem_s, accc, onesv) = refs
    else:
      (x_r, src_h, dst_h,
       out_s, acc, srcb, dstb, rows, sem, sem_s) = refs
    c = lax.axis_index("c")
    s = lax.axis_index("s")
    wid = s * NC + c
    r0 = s * RPT

    zero16 = jnp.zeros((16,), jnp.float32)

    def zfill(i, carry):
      for k2 in range(D // 16):
        rows[0, i, pl.ds(k2 * 16, 16)] = zero16
      return carry

    lax.fori_loop(0, CH, zfill, 0)

    def zcp(i, carry):
      pltpu.sync_copy(rows.at[0], acc.at[pl.ds(r0 + i * CH, CH)])
      return carry

    lax.fori_loop(0, RPT // CH, zcp, 0)
    pltpu.sync_copy(src_h.at[wid], srcb)
    pltpu.sync_copy(dst_h.at[wid], dstb)
    if do_cnt:
      pltpu.sync_copy(zc, accc.at[pl.ds(r0, RPT)])
      pltpu.sync_copy(on, onesv)
    plsc.subcore_barrier()

    pltpu.async_copy(x_r.at[srcb.at[0]], rows.at[0], sem)

    def step(j, carry):
      jm = j % 2
      pltpu.make_async_copy(x_r.at[srcb.at[j]], rows.at[jm], sem).wait()
      pltpu.async_copy(rows.at[jm], acc.at[dstb.at[j]], sem_s, add=True)
      if do_cnt:
        pltpu.sync_copy(onesv, accc.at[dstb.at[j]], add=True)

      @pl.when(j >= 1)
      def _():
        pltpu.make_async_copy(rows.at[1 - jm], acc.at[dstb.at[j]],
                              sem_s).wait()

      @pl.when(j + 1 < NCHUNK)
      def _():
        pltpu.async_copy(x_r.at[srcb.at[j + 1]], rows.at[1 - jm], sem)

      return carry

    lax.fori_loop(0, NCHUNK, step, 0)
    pltpu.make_async_copy(rows.at[(NCHUNK - 1) % 2], acc.at[dstb.at[0]],
                          sem_s).wait()
    plsc.subcore_barrier()
    pltpu.sync_copy(acc.at[pl.ds(r0, RPT)], out_s.at[c, pl.ds(r0, RPT)])
    if do_cnt:
      pltpu.sync_copy(accc.at[pl.ds(r0, RPT)], out_c.at[c, pl.ds(r0, RPT)])

  out_type = [jax.ShapeDtypeStruct((NC, NPAD, D), jnp.float32)]
  scratch = [
      pltpu.VMEM_SHARED((NPAD, D), jnp.float32),
      pltpu.VMEM((NCHUNK, CH), jnp.int32),
      pltpu.VMEM((NCHUNK, CH), jnp.int32),
      pltpu.VMEM((2, CH, D), jnp.float32),
      pltpu.SemaphoreType.DMA,
      pltpu.SemaphoreType.DMA,
  ]
  args = [x, src_r, dst_r]
  if do_cnt:
    out_type.append(jax.ShapeDtypeStruct((NC, NPAD, CW), jnp.float32))
    scratch += [pltpu.VMEM_SHARED((NPAD, CW), jnp.float32),
                pltpu.VMEM((CH, CW), jnp.float32)]
    args += [zcnt, ones]

  kern = pl.kernel(body, out_type=tuple(out_type), mesh=_mesh(),
                   compiler_params=_SC_PARAMS, scratch_types=scratch)
  return kern(*args)


def _tc_layer(x, s2, c2, ws, wn, pp):
  Din = x.shape[1]

  def body(xr, sr, cr, wsr, wnr, ppr, hr, gr):
    sv = sr[...]
    cv = cr[...]
    cnt = cv[0][:, 0:1] + cv[1][:, 0:1]
    hn = (sv[0] + sv[1]) / jnp.maximum(cnt, 1.0)
    r = jnp.dot(xr[...], wsr[...], preferred_element_type=jnp.float32)
    r = r + jnp.dot(hn, wnr[...], preferred_element_type=jnp.float32)
    r = r + ppr[0:1, :]
    r = jnp.where(r > 0, r, ppr[1:2, :] * r)
    r = ppr[2:3, :] * r + ppr[3:4, :]
    hr[...] = r

    @pl.when(pl.program_id(0) == 0)
    def _():
      gr[...] = jnp.zeros_like(gr)

    gr[...] += jnp.sum(r, axis=0, keepdims=True) * (1.0 / N)

  return pl.pallas_call(
      body,
      grid=(N // BLK,),
      in_specs=[
          pl.BlockSpec((BLK, Din), lambda i: (i, 0)),
          pl.BlockSpec((NC, BLK, Din), lambda i: (0, i, 0)),
          pl.BlockSpec((NC, BLK, CW), lambda i: (0, i, 0)),
          pl.BlockSpec((Din, H), lambda i: (0, 0)),
          pl.BlockSpec((Din, H), lambda i: (0, 0)),
          pl.BlockSpec((8, H), lambda i: (0, 0)),
      ],
      out_specs=[
          pl.BlockSpec((BLK, H), lambda i: (i, 0)),
          pl.BlockSpec((1, H), lambda i: (0, 0)),
      ],
      out_shape=[
          jax.ShapeDtypeStruct((N, H), jnp.float32),
          jax.ShapeDtypeStruct((1, H), jnp.float32),
      ],
  )(x, s2, c2, ws, wn, pp)


def kernel(feat_h, feat_p, feat_hp, eidx_h, eidx_p, eidx_hp,
           emb_h, emb_p, emb_hp, Ws1, Wn1, b1, a1, g1, be1,
           Ws, Wn, b, a, g, be):
  f = jnp.stack([feat_h, feat_p, feat_hp]).astype(jnp.int32)
  f = jnp.pad(f, ((0, 0), (0, N_PADL - N)))
  f_r = f.reshape(3, NW, NLC, LC)

  srcs, dsts = [], []
  for eidx in (eidx_h, eidx_p, eidx_hp):
    ei = eidx.astype(jnp.int32)
    srcp = jnp.pad(ei[0], (0, E_PAD - E))
    dstp = jnp.pad(ei[1], (0, E_PAD - E), constant_values=N)
    srcs.append(srcp.reshape(NW, NCHUNK, CH))
    dsts.append(dstp.reshape(NW, NCHUNK, CH))

  zcnt = jnp.zeros((RPT, CW), jnp.float32)
  ones = jnp.ones((CH, CW), jnp.float32)

  x0 = _emb_lookup(emb_h, emb_p, emb_hp, f_r)

  def pack_params(bb, aa, gg, bee):
    return jnp.concatenate(
        [jnp.stack([bb, aa, gg, bee]), jnp.zeros((4, H), jnp.float32)], axis=0)

  gsums = []
  for t in range(3):
    xt = x0[t]
    s2, c2 = _seg_sum(EMB, True, xt, srcs[t], dsts[t], zcnt, ones)
    h, gs = _tc_layer(xt, s2, c2, Ws1[t], Wn1[t],
                      pack_params(b1[t], a1[t], g1[t], be1[t]))
    t_gs = [gs]
    for l in range(3):
      (s2,) = _seg_sum(H, False, h, srcs[t], dsts[t])
      h, gs = _tc_layer(h, s2, c2, Ws[l, t], Wn[l, t],
                        pack_params(b[l, t], a[l, t], g[l, t], be[l, t]))
      t_gs.append(gs)
    gsums.append(t_gs)

  g_vec = jnp.concatenate(
      [gsums[t][l].reshape(H) for t in range(3) for l in range(4)])
  return g_vec.reshape(1, 12 * H)


# speedup vs baseline: 15.0788x; 1.1168x over previous
import functools

import jax
import jax.numpy as jnp
from jax import lax
from jax.experimental import pallas as pl
from jax.experimental.pallas import tpu as pltpu
from jax.experimental.pallas import tpu_sc as plsc

N = 10000
E = 160000
EMB = 64
H = 128
VOCAB = 257

NC = 2
NS = 16
NW = NC * NS
CH = 128
NCHUNK = -(-E // (NW * CH))
EPW = NCHUNK * CH
E_PAD = EPW * NW
NPAD = 10240
RPT = NPAD // NS
LC = 80
NLC = 4
LPW = NLC * LC
N_PADL = LPW * NW
CW = 8
BLK = 1000


def _mesh():
  return plsc.VectorSubcoreMesh(core_axis_name="c", subcore_axis_name="s")


_SC_PARAMS = pltpu.CompilerParams(use_tc_tiling_on_sc=False)


def _emb_lookup(e0, e1, e2, f_r):
  def body(e0r, e1r, e2r, fr, out, idxb, rowsb, sem):
    c = lax.axis_index("c")
    s = lax.axis_index("s")
    wid = s * NC + c
    base = wid * LPW
    for t, et in enumerate((e0r, e1r, e2r)):
      pltpu.sync_copy(fr.at[t, wid], idxb)
      for j in range(NLC):
        pltpu.async_copy(et.at[idxb.at[j]], rowsb, sem).wait()
        pltpu.sync_copy(rowsb, out.at[t, pl.ds(base + j * LC, LC)])

  kern = pl.kernel(
      body,
      out_type=jax.ShapeDtypeStruct((3, N_PADL, EMB), jnp.float32),
      mesh=_mesh(),
      compiler_params=_SC_PARAMS,
      scratch_types=[
          pltpu.VMEM((NLC, LC), jnp.int32),
          pltpu.VMEM((LC, EMB), jnp.float32),
          pltpu.SemaphoreType.DMA,
      ],
  )
  return kern(e0, e1, e2, f_r)


def _seg_sum3(D, do_cnt, x3, src_r, dst_r, zcnt=None, ones=None):
  def body(*refs):
    if do_cnt:
      (x_r, src_h, dst_h, zc, on, out_s, out_c,
       acc, srcb, dstb, rows, sem, accc, onesv) = refs
    else:
      (x_r, src_h, dst_h,
       out_s, acc, srcb, dstb, rows, sem) = refs
    c = lax.axis_index("c")
    s = lax.axis_index("s")
    wid = s * NC + c
    r0 = s * RPT

    zero16 = jnp.zeros((16,), jnp.float32)
    if do_cnt:
      pltpu.sync_copy(on, onesv)

    for t in range(3):
      def zfill(i, carry):
        for k2 in range(D // 16):
          rows[0, i, pl.ds(k2 * 16, 16)] = zero16
        return carry

      lax.fori_loop(0, CH, zfill, 0)

      def zcp(i, carry):
        pltpu.sync_copy(rows.at[0], acc.at[pl.ds(r0 + i * CH, CH)])
        return carry

      lax.fori_loop(0, RPT // CH, zcp, 0)
      pltpu.sync_copy(src_h.at[t, wid], srcb)
      pltpu.sync_copy(dst_h.at[t, wid], dstb)
      if do_cnt:
        pltpu.sync_copy(zc, accc.at[pl.ds(r0, RPT)])
      plsc.subcore_barrier()

      x_rt = x_r.at[t]
      pltpu.async_copy(x_rt.at[srcb.at[0]], rows.at[0], sem)

      def step(j, carry):
        nxt = j + 1

        @pl.when(nxt < NCHUNK)
        def _():
          pltpu.async_copy(x_rt.at[srcb.at[nxt]], rows.at[nxt % 2], sem)

        pltpu.make_async_copy(x_rt.at[srcb.at[j]], rows.at[j % 2],
                              sem).wait()
        pltpu.sync_copy(rows.at[j % 2], acc.at[dstb.at[j]], add=True)
        if do_cnt:
          pltpu.sync_copy(onesv, accc.at[dstb.at[j]], add=True)
        return carry

      lax.fori_loop(0, NCHUNK, step, 0)
      plsc.subcore_barrier()
      pltpu.sync_copy(acc.at[pl.ds(r0, RPT)],
                      out_s.at[t, c, pl.ds(r0, RPT)])
      if do_cnt:
        pltpu.sync_copy(accc.at[pl.ds(r0, RPT)],
                        out_c.at[t, c, pl.ds(r0, RPT)])

  out_type = [jax.ShapeDtypeStruct((3, NC, NPAD, D), jnp.float32)]
  scratch = [
      pltpu.VMEM_SHARED((NPAD, D), jnp.float32),
      pltpu.VMEM((NCHUNK, CH), jnp.int32),
      pltpu.VMEM((NCHUNK, CH), jnp.int32),
      pltpu.VMEM((2, CH, D), jnp.float32),
      pltpu.SemaphoreType.DMA,
  ]
  args = [x3, src_r, dst_r]
  if do_cnt:
    out_type.append(jax.ShapeDtypeStruct((3, NC, NPAD, CW), jnp.float32))
    scratch += [pltpu.VMEM_SHARED((NPAD, CW), jnp.float32),
                pltpu.VMEM((CH, CW), jnp.float32)]
    args += [zcnt, ones]

  kern = pl.kernel(body, out_type=tuple(out_type), mesh=_mesh(),
                   compiler_params=_SC_PARAMS, scratch_types=scratch)
  return kern(*args)


def _tc_layer3(x3, s3, c3, ws3, wn3, pp3):
  Din = x3.shape[2]

  def body(xr, sr, cr, wsr, wnr, ppr, hr, gr):
    sv = sr[0]
    cv = cr[0]
    cnt = cv[0][:, 0:1] + cv[1][:, 0:1]
    hn = (sv[0] + sv[1]) / jnp.maximum(cnt, 1.0)
    r = jnp.dot(xr[0], wsr[0], preferred_element_type=jnp.float32)
    r = r + jnp.dot(hn, wnr[0], preferred_element_type=jnp.float32)
    r = r + ppr[0, 0:1, :]
    r = jnp.where(r > 0, r, ppr[0, 1:2, :] * r)
    r = ppr[0, 2:3, :] * r + ppr[0, 3:4, :]
    hr[0] = r

    @pl.when(pl.program_id(1) == 0)
    def _():
      gr[0] = jnp.zeros_like(gr[0])

    gr[0] += jnp.sum(r, axis=0, keepdims=True) * (1.0 / N)

  return pl.pallas_call(
      body,
      grid=(3, N // BLK),
      in_specs=[
          pl.BlockSpec((1, BLK, Din), lambda t, i: (t, i, 0)),
          pl.BlockSpec((1, NC, BLK, Din), lambda t, i: (t, 0, i, 0)),
          pl.BlockSpec((1, NC, BLK, CW), lambda t, i: (t, 0, i, 0)),
          pl.BlockSpec((1, Din, H), lambda t, i: (t, 0, 0)),
          pl.BlockSpec((1, Din, H), lambda t, i: (t, 0, 0)),
          pl.BlockSpec((1, 8, H), lambda t, i: (t, 0, 0)),
      ],
      out_specs=[
          pl.BlockSpec((1, BLK, H), lambda t, i: (t, i, 0)),
          pl.BlockSpec((1, 1, H), lambda t, i: (t, 0, 0)),
      ],
      out_shape=[
          jax.ShapeDtypeStruct((3, N, H), jnp.float32),
          jax.ShapeDtypeStruct((3, 1, H), jnp.float32),
      ],
  )(x3, s3, c3, ws3, wn3, pp3)


def kernel(feat_h, feat_p, feat_hp, eidx_h, eidx_p, eidx_hp,
           emb_h, emb_p, emb_hp, Ws1, Wn1, b1, a1, g1, be1,
           Ws, Wn, b, a, g, be):
  f = jnp.stack([feat_h, feat_p, feat_hp]).astype(jnp.int32)
  f = jnp.pad(f, ((0, 0), (0, N_PADL - N)))
  f_r = f.reshape(3, NW, NLC, LC)

  srcs, dsts = [], []
  for eidx in (eidx_h, eidx_p, eidx_hp):
    ei = eidx.astype(jnp.int32)
    srcs.append(jnp.pad(ei[0], (0, E_PAD - E)))
    dsts.append(jnp.pad(ei[1], (0, E_PAD - E), constant_values=N))
  src3 = jnp.stack(srcs).reshape(3, NW, NCHUNK, CH)
  dst3 = jnp.stack(dsts).reshape(3, NW, NCHUNK, CH)

  zcnt = jnp.zeros((RPT, CW), jnp.float32)
  ones = jnp.ones((CH, CW), jnp.float32)

  x3 = _emb_lookup(emb_h, emb_p, emb_hp, f_r)

  def pack_params(bb, aa, gg, bee):
    p4 = jnp.stack([bb, aa, gg, bee], axis=1)
    return jnp.concatenate([p4, jnp.zeros((3, 4, H), jnp.float32)], axis=1)

  s3, c3 = _seg_sum3(EMB, True, x3, src3, dst3, zcnt, ones)
  h3, gv = _tc_layer3(x3, s3, c3, Ws1, Wn1, pack_params(b1, a1, g1, be1))
  gs = [gv]
  for l in range(3):
    (s3,) = _seg_sum3(H, False, h3, src3, dst3)
    h3, gv = _tc_layer3(h3, s3, c3, Ws[l], Wn[l],
                        pack_params(b[l], a[l], g[l], be[l]))
    gs.append(gv)

  g_vec = jnp.concatenate(
      [gs[l][t].reshape(H) for t in range(3) for l in range(4)])
  return g_vec.reshape(1, 12 * H)
